# baseline (speedup 1.0000x reference)
import jax
import jax.numpy as jnp
from jax import lax
from jax.experimental import pallas as pl
from jax.experimental.pallas import tpu as pltpu
from jax.experimental.pallas import tpu_sc as plsc

DIM = 32
NROW = 4096
NCOL = 200

_info = plsc.get_sparse_core_info()
NC = _info.num_cores
NS = _info.num_subcores
NW = NC * NS

RT = NCOL // 8
CT = NROW // 128
ST = DIM // 8

CHUNK = 1024
TPC = CHUNK // 128
NCH = (RT * 8 * 128) // CHUNK


def _gather_body(idxp_hbm, table_hbm, out_hbm,
                 idx_v, rows_v, outt_v, gsem, ssem0, ssem1):
    ct = lax.axis_index("s") * NC + lax.axis_index("c")
    ssem = (ssem0, ssem1)

    pltpu.sync_copy(idxp_hbm.at[:, ct], idx_v)

    lane = lax.iota(jnp.int32, 16)
    base = (lane * 128, lane * 128 + 16 * 128)
    G = 8
    SLC = DIM * 128 - 128 + G

    def start_gather(c, b):
        pltpu.make_async_copy(
            table_hbm.at[idx_v.at[c]],
            rows_v.at[pl.ds(pl.multiple_of(b * CHUNK, 8), CHUNK), :],
            gsem,
        ).start()

    def wait_gather():
        pltpu.make_async_copy(
            table_hbm.at[idx_v.at[0]],
            rows_v.at[pl.ds(0, CHUNK), :],
            gsem,
        ).wait()

    def start_store(t, o):
        for st in range(ST):
            pltpu.make_async_copy(
                outt_v.at[pl.ds(pl.multiple_of(o * (DIM * 128) + st * 1024, 8),
                                1024)],
                out_hbm.at[t, st, ct],
                ssem[o],
            ).start()

    def wait_store(o):
        for st in range(ST):
            pltpu.make_async_copy(
                outt_v.at[pl.ds(st * 1024, 1024)],
                out_hbm.at[0, st, ct],
                ssem[o],
            ).wait()

    def transpose(row0, obase):
        def loads(i0):
            return [rows_v[row0 + i0 + k, pl.ds(h * 16, 16)]
                    for k in range(G) for h in range(2)]

        def stores(i0, xs):
            off = pl.multiple_of(obase + i0, 8)
            for k in range(G):
                for h in range(2):
                    plsc.store_scatter(
                        outt_v.at[pl.ds(off, SLC)], [base[h] + k],
                        xs[k * 2 + h],
                    )

        xs = loads(0)
        for i0 in range(G, 128, G):
            ys = loads(i0)
            stores(i0 - G, xs)
            xs = ys
        stores(128 - G, xs)

    start_gather(0, 0)

    def chunk(c, carry):
        b = lax.rem(c, 2)
        wait_gather()
        start_gather(lax.rem(c + 1, NCH), lax.rem(c + 1, 2))

        def tile(ts, carry2):
            t = c * TPC + ts
            o = lax.rem(t, 2)

            @pl.when(t >= 2)
            def _():
                @pl.when(o == 0)
                def _():
                    wait_store(0)

                @pl.when(o == 1)
                def _():
                    wait_store(1)

            transpose(b * CHUNK + ts * 128, o * (DIM * 128))

            @pl.when(o == 0)
            def _():
                start_store(t, 0)

            @pl.when(o == 1)
            def _():
                start_store(t, 1)

            return carry2

        lax.fori_loop(0, TPC, tile, 0)
        return carry

    lax.fori_loop(0, NCH, chunk, 0)

    wait_gather()
    wait_store(0)
    wait_store(1)


_mesh = plsc.VectorSubcoreMesh(core_axis_name="c", subcore_axis_name="s")

_gather = pl.kernel(
    _gather_body,
    out_type=jax.ShapeDtypeStruct((NCOL, ST, CT, 1024), jnp.float32),
    mesh=_mesh,
    scratch_types=[
        pltpu.VMEM((RT, CHUNK), jnp.int32),
        pltpu.VMEM((2 * CHUNK, DIM), jnp.float32),
        pltpu.VMEM((2 * DIM * 128,), jnp.float32),
        pltpu.SemaphoreType.DMA,
        pltpu.SemaphoreType.DMA,
        pltpu.SemaphoreType.DMA,
    ],
    compiler_params=pltpu.CompilerParams(
        use_tc_tiling_on_sc=False, needs_layout_passes=False
    ),
)


@jax.jit
def kernel(data, table):
    dataP = data.T.reshape(RT, 8, CT, 128).transpose(0, 2, 1, 3).reshape(RT, CT, 1024)
    o4 = _gather(dataP.astype(jnp.int32), table)
    o5 = o4.reshape(NCOL, ST, CT, 8, 128)
    out = o5.transpose(2, 4, 0, 1, 3).reshape(NROW, NCOL, DIM)
    return out


# speedup vs baseline: 1.5984x; 1.5984x over previous
import jax
import jax.numpy as jnp
from jax import lax
from jax.experimental import pallas as pl
from jax.experimental.pallas import tpu as pltpu
from jax.experimental.pallas import tpu_sc as plsc

DIM = 32
NROW = 4096
NCOL = 200

_info = plsc.get_sparse_core_info()
NC = _info.num_cores
NS = _info.num_subcores
NW = NC * NS

RT = NCOL // 8
CT = NROW // 128
ST = DIM // 8

CHUNK = 1024
TPC = CHUNK // 128
NCH = (RT * 8 * 128) // CHUNK


def _gather_body(idxp_hbm, table_hbm, out_hbm,
                 idx_v, rows_v, outt_v, gsem, ssem0, ssem1):
    ct = lax.axis_index("s") * NC + lax.axis_index("c")
    ssem = (ssem0, ssem1)

    pltpu.sync_copy(idxp_hbm.at[:, ct], idx_v)

    lane = lax.iota(jnp.int32, 16)

    def start_gather(c, b):
        pltpu.make_async_copy(
            table_hbm.at[idx_v.at[c]],
            rows_v.at[pl.ds(pl.multiple_of(b * CHUNK, 8), CHUNK), :],
            gsem,
        ).start()

    def wait_gather():
        pltpu.make_async_copy(
            table_hbm.at[idx_v.at[0]],
            rows_v.at[pl.ds(0, CHUNK), :],
            gsem,
        ).wait()

    def start_store(t, o):
        for st in range(ST):
            pltpu.make_async_copy(
                outt_v.at[pl.ds(pl.multiple_of(o * (DIM * 128) + st * 1024, 8),
                                1024)],
                out_hbm.at[t, st, ct],
                ssem[o],
            ).start()

    def wait_store(o):
        for st in range(ST):
            pltpu.make_async_copy(
                outt_v.at[pl.ds(st * 1024, 1024)],
                out_hbm.at[0, st, ct],
                ssem[o],
            ).wait()

    W = 4

    def transpose(row0, obase):
        off = pl.multiple_of(obase, 8)
        dst = outt_v.at[pl.ds(off, DIM * 128)]

        def block(i0, carry):
            for h in range(2):
                sv = lane + h * 16
                sv128 = sv * 128 + i0

                def rot(j):
                    return (lane + j) & 15

                def load(j):
                    return plsc.load_gather(rows_v, [row0 + i0 + rot(j), sv])

                xs = [load(j) for j in range(W)]
                for j in range(16):
                    if j + W < 16:
                        xs.append(load(j + W))
                    plsc.store_scatter(dst, [sv128 + rot(j)], xs[j])
            return carry

        lax.fori_loop(0, 128 // 16, lambda i, c: block(i * 16, c), 0)

    start_gather(0, 0)

    def chunk(c, carry):
        b = lax.rem(c, 2)
        wait_gather()
        start_gather(lax.rem(c + 1, NCH), lax.rem(c + 1, 2))

        def tile(ts, carry2):
            t = c * TPC + ts
            o = lax.rem(t, 2)

            @pl.when(t >= 2)
            def _():
                @pl.when(o == 0)
                def _():
                    wait_store(0)

                @pl.when(o == 1)
                def _():
                    wait_store(1)

            transpose(b * CHUNK + ts * 128, o * (DIM * 128))

            @pl.when(o == 0)
            def _():
                start_store(t, 0)

            @pl.when(o == 1)
            def _():
                start_store(t, 1)

            return carry2

        lax.fori_loop(0, TPC, tile, 0)
        return carry

    lax.fori_loop(0, NCH, chunk, 0)

    wait_gather()
    wait_store(0)
    wait_store(1)


_mesh = plsc.VectorSubcoreMesh(core_axis_name="c", subcore_axis_name="s")

_gather = pl.kernel(
    _gather_body,
    out_type=jax.ShapeDtypeStruct((NCOL, ST, CT, 1024), jnp.float32),
    mesh=_mesh,
    scratch_types=[
        pltpu.VMEM((RT, CHUNK), jnp.int32),
        pltpu.VMEM((2 * CHUNK, DIM), jnp.float32),
        pltpu.VMEM((2 * DIM * 128,), jnp.float32),
        pltpu.SemaphoreType.DMA,
        pltpu.SemaphoreType.DMA,
        pltpu.SemaphoreType.DMA,
    ],
    compiler_params=pltpu.CompilerParams(
        use_tc_tiling_on_sc=False, needs_layout_passes=False
    ),
)


@jax.jit
def kernel(data, table):
    dataP = data.T.reshape(RT, 8, CT, 128).transpose(0, 2, 1, 3).reshape(RT, CT, 1024)
    o4 = _gather(dataP.astype(jnp.int32), table)
    o5 = o4.reshape(NCOL, ST, CT, 8, 128)
    out = o5.transpose(2, 4, 0, 1, 3).reshape(NROW, NCOL, DIM)
    return out
